# baseline (speedup 1.0000x reference)
import jax
import jax.numpy as jnp
from jax import lax
from jax.experimental import pallas as pl
from jax.experimental.pallas import tpu as pltpu
from jax.experimental.pallas import tpu_sc as plsc

N = 4096
BLOCK_R = 256
NUM_CORES = 2
NUM_SUBCORES = 16
NUM_WORKERS = NUM_CORES * NUM_SUBCORES
ROWS_PER_WORKER = N // NUM_WORKERS
LANES = 16


def _tc_zero_kernel(out_ref):
    out_ref[:, :] = jnp.zeros((BLOCK_R, N), jnp.float32)


def _tc_zeros():
    return pl.pallas_call(
        _tc_zero_kernel,
        grid=(N // BLOCK_R,),
        out_specs=pl.BlockSpec((BLOCK_R, N), lambda i: (i, 0)),
        out_shape=jax.ShapeDtypeStruct((N, N), jnp.float32),
    )()


def _sc_scatter_body(perm_hbm, mat, idx_v, src, sem):
    c = lax.axis_index("c")
    s = lax.axis_index("s")
    wid = s * NUM_CORES + c
    base = wid * ROWS_PER_WORKER

    pltpu.sync_copy(perm_hbm.at[pl.ds(base, ROWS_PER_WORKER)], idx_v)

    zeros = jnp.zeros((LANES,), jnp.float32)
    ones = jnp.ones((LANES,), jnp.float32)
    lanes = lax.iota(jnp.int32, LANES)

    def _zero_row(r, _):
        src[r, :] = zeros
        return 0

    lax.fori_loop(0, ROWS_PER_WORKER, _zero_row, 0, unroll=4)

    for st in range(ROWS_PER_WORKER // LANES):
        cols = idx_v[pl.ds(st * LANES, LANES)]
        rows = st * LANES + lanes
        plsc.store_scatter(src, [rows, jnp.bitwise_and(cols, LANES - 1)], ones)

    copies = []
    neg = jnp.full((LANES,), jnp.int32(-1))
    for r in range(ROWS_PER_WORKER):
        chunk = idx_v[pl.ds((r // LANES) * LANES, LANES)]
        col = jnp.max(jnp.where(lanes == (r % LANES), chunk, neg))
        grp = pl.multiple_of(jnp.bitwise_and(col, jnp.int32(~(LANES - 1))), LANES)
        copies.append(
            pltpu.make_async_copy(
                src.at[r], mat.at[base + r, pl.ds(grp, LANES)], sem
            )
        )
        copies[-1].start()
    for cp in copies:
        cp.wait()


def _sc_scatter(mat_ref, perm):
    mesh = plsc.VectorSubcoreMesh(
        core_axis_name="c", subcore_axis_name="s",
        num_cores=NUM_CORES, num_subcores=NUM_SUBCORES,
    )
    return pl.kernel(
        _sc_scatter_body,
        mesh=mesh,
        scratch_types=[
            pltpu.VMEM((ROWS_PER_WORKER,), jnp.int32),
            pltpu.VMEM((ROWS_PER_WORKER, LANES), jnp.float32),
            pltpu.SemaphoreType.DMA,
        ],
        compiler_params=pltpu.CompilerParams(
            needs_layout_passes=False, use_tc_tiling_on_sc=False
        ),
    )(perm, mat_ref)


def kernel(perm):
    perm = perm.astype(jnp.int32)
    mat_ref = jax.new_ref(_tc_zeros())
    _sc_scatter(mat_ref, perm)
    return mat_ref[...]


# speedup vs baseline: 4.0266x; 4.0266x over previous
import functools

import jax
import jax.numpy as jnp
from jax import lax
from jax.experimental import pallas as pl
from jax.experimental.pallas import tpu as pltpu
from jax.experimental.pallas import tpu_sc as plsc

N = 4096
NUM_CORES = 2
NUM_SUBCORES = 16
NUM_WORKERS = NUM_CORES * NUM_SUBCORES
ROWS_PER_WORKER = N // NUM_WORKERS
CHUNK = 8
STEPS = ROWS_PER_WORKER // CHUNK
LANES = 16


def _sc_body(perm_hbm, out_hbm, idx_v, buf0, buf1, sem0, sem1):
    c = lax.axis_index("c")
    s = lax.axis_index("s")
    wid = s * NUM_CORES + c
    base = wid * ROWS_PER_WORKER

    pltpu.sync_copy(perm_hbm.at[pl.ds(base, ROWS_PER_WORKER)], idx_v)

    zeros = jnp.zeros((LANES,), jnp.float32)
    ones = jnp.ones((LANES,), jnp.float32)
    lanes = lax.iota(jnp.int32, LANES)
    lo = lanes < CHUNK

    bufs = (buf0, buf1)
    sems = (sem0, sem1)

    def _zero_cols(j, _):
        for r in range(CHUNK):
            buf0[r, pl.ds(j * LANES, LANES)] = zeros
            buf1[r, pl.ds(j * LANES, LANES)] = zeros
        return 0

    lax.fori_loop(0, N // LANES, _zero_cols, 0, unroll=4)

    def _cols_at(st):
        off = min(st * CHUNK, ROWS_PER_WORKER - LANES)
        shift = st * CHUNK - off
        window = idx_v[pl.ds(off, LANES)]
        return window, shift

    def _prep(b, st):
        window, shift = _cols_at(st)
        rows = lanes - shift
        mask = (rows >= 0) & (rows < CHUNK)
        plsc.store_scatter(bufs[b], [rows, window], ones, mask=mask)

    def _clear(b, st):
        window, shift = _cols_at(st)
        rows = lanes - shift
        mask = (rows >= 0) & (rows < CHUNK)
        plsc.store_scatter(bufs[b], [rows, window], zeros, mask=mask)

    def _send(b, st):
        return pltpu.make_async_copy(
            bufs[b], out_hbm.at[pl.ds(base + st * CHUNK, CHUNK)], sems[b]
        )

    inflight = [None, None]
    for st in range(STEPS):
        b = st & 1
        if inflight[b] is not None:
            inflight[b].wait()
            _clear(b, st - 2)
        _prep(b, st)
        dma = _send(b, st)
        dma.start()
        inflight[b] = dma
    for b in (0, 1):
        if inflight[b] is not None:
            inflight[b].wait()


@functools.partial(jax.jit, static_argnums=())
def _sc_build(perm):
    mesh = plsc.VectorSubcoreMesh(
        core_axis_name="c", subcore_axis_name="s",
        num_cores=NUM_CORES, num_subcores=NUM_SUBCORES,
    )
    return pl.kernel(
        _sc_body,
        out_type=jax.ShapeDtypeStruct((N, N), jnp.float32),
        mesh=mesh,
        scratch_types=[
            pltpu.VMEM((ROWS_PER_WORKER,), jnp.int32),
            pltpu.VMEM((CHUNK, N), jnp.float32),
            pltpu.VMEM((CHUNK, N), jnp.float32),
            pltpu.SemaphoreType.DMA,
            pltpu.SemaphoreType.DMA,
        ],
        compiler_params=pltpu.CompilerParams(needs_layout_passes=False),
    )(perm)


def kernel(perm):
    return _sc_build(perm.astype(jnp.int32))
